# baseline (speedup 1.0000x reference)
import jax
import jax.numpy as jnp
from jax import lax
from jax.experimental import pallas as pl
from jax.experimental.pallas import tpu as pltpu
from jax.experimental.pallas import tpu_sc as plsc

_NC, _NS, _L = 2, 16, 16
_NW = _NC * _NS
_TBL = 1024


def _body(t_hbm, tbl_hbm, out_b_hbm, out_a_hbm,
          idx_v, tbl_v, ob_v, oa_v, sem_in, sem_out):
    wid = lax.axis_index("s") * _NC + lax.axis_index("c")
    bw = idx_v.shape[0]
    base = wid * bw
    cp_t = pltpu.async_copy(t_hbm.at[pl.ds(base, bw)], idx_v, sem_in)
    cp_tb = pltpu.async_copy(tbl_hbm, tbl_v, sem_in)
    cp_t.wait()
    cp_tb.wait()

    def step(i, carry):
        off = i * _L
        idx = idx_v[pl.ds(off, _L)]
        ob_v[pl.ds(off, _L)] = plsc.load_gather(tbl_v, [idx])
        oa_v[pl.ds(off, _L)] = plsc.load_gather(tbl_v, [idx + _TBL])
        return carry

    lax.fori_loop(0, bw // _L, step, 0)
    co_b = pltpu.async_copy(ob_v, out_b_hbm.at[pl.ds(base, bw)], sem_out)
    co_a = pltpu.async_copy(oa_v, out_a_hbm.at[pl.ds(base, bw)], sem_out)
    co_b.wait()
    co_a.wait()


def kernel(t, beta, alpha):
    b = t.shape[0]
    bw = b // _NW
    n = beta.shape[0]
    tbl = jnp.concatenate([jnp.pad(beta, (0, _TBL - n)),
                           jnp.pad(alpha, (0, _TBL - n))])
    run = pl.kernel(
        _body,
        out_type=(jax.ShapeDtypeStruct((b,), jnp.float32),
                  jax.ShapeDtypeStruct((b,), jnp.float32)),
        mesh=plsc.VectorSubcoreMesh(core_axis_name="c", subcore_axis_name="s"),
        scratch_types=[
            pltpu.VMEM((bw,), jnp.int32),
            pltpu.VMEM((2 * _TBL,), jnp.float32),
            pltpu.VMEM((bw,), jnp.float32),
            pltpu.VMEM((bw,), jnp.float32),
            pltpu.SemaphoreType.DMA,
            pltpu.SemaphoreType.DMA,
        ],
        compiler_params=pltpu.CompilerParams(needs_layout_passes=False),
    )
    return run(t, tbl)
